# Initial kernel scaffold; baseline (speedup 1.0000x reference)
import functools

import jax
import jax.numpy as jnp
from jax import lax
from jax.experimental import pallas as pl
from jax.experimental.pallas import tpu as pltpu
from jax.experimental.pallas import tpu_sc as plsc

N_VOCAB = 100000
N_EMBED = 64
BATCH = 16384
N_SAMPLES = 5

NC = 2
NS = 16
NW = NC * NS

CHUNK = 512

PER_W_A = BATCH // NW
PER_W_B = BATCH // NW
PER_W_C = BATCH * N_SAMPLES // NW
TOTAL = BATCH * (2 + N_SAMPLES)


def _gather_body(in_idx_hbm, out_idx_hbm, noise_idx_hbm, in_tab_hbm,
                 out_tab_hbm, out_hbm, idx_v, rows_v, sem):
    wid = lax.axis_index("s") * NC + lax.axis_index("c")

    def one_chunk(idx_src, idx_off, table, dst_off):
        pltpu.sync_copy(idx_src.at[pl.ds(idx_off, CHUNK)], idx_v)
        pltpu.async_copy(table.at[idx_v], rows_v, sem).wait()
        pltpu.sync_copy(rows_v, out_hbm.at[pl.ds(dst_off, CHUNK)])

    one_chunk(in_idx_hbm, wid * PER_W_A, in_tab_hbm, wid * PER_W_A)
    one_chunk(out_idx_hbm, wid * PER_W_B, out_tab_hbm, BATCH + wid * PER_W_B)
    for c in range(PER_W_C // CHUNK):
        off = wid * PER_W_C + c * CHUNK
        one_chunk(noise_idx_hbm, off, out_tab_hbm, 2 * BATCH + off)


def kernel(input_words, output_words, noise_words, in_embed_weight,
           out_embed_weight):
    mesh = plsc.VectorSubcoreMesh(core_axis_name="c", subcore_axis_name="s")
    f = pl.kernel(
        _gather_body,
        mesh=mesh,
        out_type=jax.ShapeDtypeStruct((TOTAL, N_EMBED), jnp.float32),
        scratch_types=[
            pltpu.VMEM((CHUNK,), jnp.int32),
            pltpu.VMEM((CHUNK, N_EMBED), jnp.float32),
            pltpu.SemaphoreType.DMA,
        ],
    )
    return f(
        input_words.astype(jnp.int32),
        output_words.astype(jnp.int32),
        noise_words.reshape(-1).astype(jnp.int32),
        in_embed_weight,
        out_embed_weight,
    )


# speedup vs baseline: 1.0807x; 1.0807x over previous
import functools

import jax
import jax.numpy as jnp
from jax import lax
from jax.experimental import pallas as pl
from jax.experimental.pallas import tpu as pltpu
from jax.experimental.pallas import tpu_sc as plsc

N_VOCAB = 100000
N_EMBED = 64
BATCH = 16384
N_SAMPLES = 5

NC = 2
NS = 16
NW = NC * NS

CHUNK = 512

PER_W_A = BATCH // NW
PER_W_B = BATCH // NW
PER_W_C = BATCH * N_SAMPLES // NW
TOTAL = BATCH * (2 + N_SAMPLES)


def _gather_body(in_idx_hbm, out_idx_hbm, noise_idx_hbm, in_tab_hbm,
                 out_tab_hbm, out_hbm, idx_v, rows_v, sem):
    wid = lax.axis_index("s") * NC + lax.axis_index("c")

    def one_chunk(idx_src, idx_off, table, dst_off):
        pltpu.sync_copy(idx_src.at[pl.ds(idx_off, CHUNK)], idx_v)
        pltpu.async_copy(table.at[idx_v], rows_v, sem).wait()
        pltpu.sync_copy(rows_v, out_hbm.at[pl.ds(dst_off, CHUNK)])

    one_chunk(in_idx_hbm, wid * PER_W_A, in_tab_hbm, wid * PER_W_A)
    one_chunk(out_idx_hbm, wid * PER_W_B, out_tab_hbm, BATCH + wid * PER_W_B)
    for c in range(PER_W_C // CHUNK):
        off = wid * PER_W_C + c * CHUNK
        one_chunk(noise_idx_hbm, off, out_tab_hbm, 2 * BATCH + off)


def kernel(input_words, output_words, noise_words, in_embed_weight,
           out_embed_weight):
    mesh = plsc.VectorSubcoreMesh(core_axis_name="c", subcore_axis_name="s")
    f = pl.kernel(
        _gather_body,
        mesh=mesh,
        out_type=jax.ShapeDtypeStruct((TOTAL, N_EMBED), jnp.float32),
        scratch_types=[
            pltpu.VMEM((CHUNK,), jnp.int32),
            pltpu.VMEM((CHUNK, N_EMBED), jnp.float32),
            pltpu.SemaphoreType.DMA,
        ],
        compiler_params=pltpu.CompilerParams(use_tc_tiling_on_sc=False),
    )
    return f(
        input_words.astype(jnp.int32),
        output_words.astype(jnp.int32),
        noise_words.reshape(-1).astype(jnp.int32),
        in_embed_weight,
        out_embed_weight,
    )


# speedup vs baseline: 1.1091x; 1.0263x over previous
import jax
import jax.numpy as jnp
from jax import lax
from jax.experimental import pallas as pl
from jax.experimental.pallas import tpu as pltpu
from jax.experimental.pallas import tpu_sc as plsc

N_VOCAB = 100000
N_EMBED = 64
BATCH = 16384
N_SAMPLES = 5

NC = 2
NS = 16
NW = NC * NS

CHUNK = 512
PER_W_A = BATCH // NW
PER_W_BC = BATCH * (1 + N_SAMPLES) // NW
PER_W = PER_W_A + PER_W_BC
N_CHUNKS = PER_W // CHUNK
NBUF = 3
TOTAL = BATCH * (2 + N_SAMPLES)


def _gather_body(in_idx_hbm, bc_idx_hbm, in_tab_hbm, out_tab_hbm, out_hbm,
                 idx_v, bufs, gsems, wsems, isem):
    wid = lax.axis_index("s") * NC + lax.axis_index("c")

    ia = pltpu.async_copy(in_idx_hbm.at[pl.ds(wid * PER_W_A, PER_W_A)],
                          idx_v.at[pl.ds(0, PER_W_A)], isem)
    ib = pltpu.async_copy(bc_idx_hbm.at[pl.ds(wid * PER_W_BC, PER_W_BC)],
                          idx_v.at[pl.ds(PER_W_A, PER_W_BC)], isem)
    ia.wait()
    ib.wait()

    def src_table(c):
        return in_tab_hbm if c == 0 else out_tab_hbm

    def dst_off(c):
        if c == 0:
            return wid * PER_W_A
        return BATCH + wid * PER_W_BC + (c - 1) * CHUNK

    def start_gather(c):
        b = c % NBUF
        return pltpu.async_copy(
            src_table(c).at[idx_v.at[pl.ds(c * CHUNK, CHUNK)]],
            bufs[b], gsems[b])

    def start_write(c):
        b = c % NBUF
        return pltpu.async_copy(
            bufs[b], out_hbm.at[pl.ds(dst_off(c), CHUNK)], wsems[b])

    gathers = [None] * N_CHUNKS
    writes = [None] * N_CHUNKS
    gathers[0] = start_gather(0)
    gathers[1] = start_gather(1)
    for c in range(N_CHUNKS):
        gathers[c].wait()
        nxt = c + 2
        if nxt < N_CHUNKS:
            if nxt - NBUF >= 0:
                writes[nxt - NBUF].wait()
            gathers[nxt] = start_gather(nxt)
        writes[c] = start_write(c)
    for c in range(N_CHUNKS - NBUF, N_CHUNKS):
        writes[c].wait()


def kernel(input_words, output_words, noise_words, in_embed_weight,
           out_embed_weight):
    bc_idx = jnp.concatenate(
        [output_words.astype(jnp.int32),
         noise_words.reshape(-1).astype(jnp.int32)], axis=0)
    mesh = plsc.VectorSubcoreMesh(core_axis_name="c", subcore_axis_name="s")
    f = pl.kernel(
        _gather_body,
        mesh=mesh,
        out_type=jax.ShapeDtypeStruct((TOTAL, N_EMBED), jnp.float32),
        scratch_types=[
            pltpu.VMEM((PER_W,), jnp.int32),
            [pltpu.VMEM((CHUNK, N_EMBED), jnp.float32)] * NBUF,
            [pltpu.SemaphoreType.DMA] * NBUF,
            [pltpu.SemaphoreType.DMA] * NBUF,
            pltpu.SemaphoreType.DMA,
        ],
        compiler_params=pltpu.CompilerParams(use_tc_tiling_on_sc=False),
    )
    return f(
        input_words.astype(jnp.int32),
        bc_idx,
        in_embed_weight,
        out_embed_weight,
    )


# speedup vs baseline: 1.9806x; 1.7857x over previous
import jax
import jax.numpy as jnp
from jax import lax
from jax.experimental import pallas as pl
from jax.experimental.pallas import tpu as pltpu
from jax.experimental.pallas import tpu_sc as plsc

N_VOCAB = 100000
N_EMBED = 64
BATCH = 16384
N_SAMPLES = 5

NC = 2
NS = 16
NW = NC * NS
DIMS_PER_W = N_EMBED // NW

TOTAL = BATCH * (2 + N_SAMPLES)
N_BC = BATCH * (1 + N_SAMPLES)

IC = 4096
UNROLL = 4


def _gather_body(in_idx_hbm, bc_idx_hbm, in_tab_t, out_tab_t, out_t,
                 row_v, idx_vs, out_vs, isems, wsems):
    wid = lax.axis_index("s") * NC + lax.axis_index("c")

    def gather_chunk(idx_v, out_v):
        def body(g, carry):
            base = g * 16 * UNROLL
            for u in range(UNROLL):
                iv = idx_v[pl.ds(base + u * 16, 16)]
                out_v[pl.ds(base + u * 16, 16)] = plsc.load_gather(row_v, [iv])
            return carry
        lax.fori_loop(0, IC // (16 * UNROLL), body, 0)

    def do_dim(j, tab, idx_hbm, idx_n, out_off):
        pltpu.sync_copy(tab.at[j], row_v)
        nch = idx_n // IC
        ids = [None, None]
        wds = [None] * nch
        ids[0] = pltpu.async_copy(idx_hbm.at[pl.ds(0, IC)], idx_vs[0],
                                  isems[0])
        for c in range(nch):
            b = c % 2
            if c + 1 < nch:
                ids[(c + 1) % 2] = pltpu.async_copy(
                    idx_hbm.at[pl.ds((c + 1) * IC, IC)],
                    idx_vs[(c + 1) % 2], isems[(c + 1) % 2])
            ids[b].wait()
            if c - 2 >= 0:
                wds[c - 2].wait()
            gather_chunk(idx_vs[b], out_vs[b])
            wds[c] = pltpu.async_copy(
                out_vs[b], out_t.at[j, pl.ds(out_off + c * IC, IC)],
                wsems[b])
        for c in range(max(0, nch - 2), nch):
            wds[c].wait()

    for t in range(DIMS_PER_W):
        j = wid * DIMS_PER_W + t
        do_dim(j, in_tab_t, in_idx_hbm, BATCH, 0)
        do_dim(j, out_tab_t, bc_idx_hbm, N_BC, BATCH)


def kernel(input_words, output_words, noise_words, in_embed_weight,
           out_embed_weight):
    bc_idx = jnp.concatenate(
        [output_words.astype(jnp.int32),
         noise_words.reshape(-1).astype(jnp.int32)], axis=0)
    mesh = plsc.VectorSubcoreMesh(core_axis_name="c", subcore_axis_name="s")
    f = pl.kernel(
        _gather_body,
        mesh=mesh,
        out_type=jax.ShapeDtypeStruct((N_EMBED, TOTAL), jnp.float32),
        scratch_types=[
            pltpu.VMEM((N_VOCAB,), jnp.float32),
            [pltpu.VMEM((IC,), jnp.int32)] * 2,
            [pltpu.VMEM((IC,), jnp.float32)] * 2,
            [pltpu.SemaphoreType.DMA] * 2,
            [pltpu.SemaphoreType.DMA] * 2,
        ],
        compiler_params=pltpu.CompilerParams(use_tc_tiling_on_sc=True,
                                             needs_layout_passes=False),
    )
    out_t = f(
        input_words.astype(jnp.int32),
        bc_idx,
        in_embed_weight.T,
        out_embed_weight.T,
    )
    return out_t.T


# speedup vs baseline: 2.2868x; 1.1546x over previous
import jax
import jax.numpy as jnp
from jax import lax
from jax.experimental import pallas as pl
from jax.experimental.pallas import tpu as pltpu
from jax.experimental.pallas import tpu_sc as plsc

N_VOCAB = 100000
N_EMBED = 64
BATCH = 16384
N_SAMPLES = 5

NC = 2
NS = 16
NW = NC * NS
DIMS_PER_W = N_EMBED // NW

TOTAL = BATCH * (2 + N_SAMPLES)
N_BC = BATCH * (1 + N_SAMPLES)

IC = 4096
UNROLL = 4


def _gather_body(in_idx_hbm, bc_idx_hbm, in_tab_t, out_tab_t, out_t,
                 row_v, idx_vs, out_vs, isems, wsems):
    wid = lax.axis_index("s") * NC + lax.axis_index("c")

    def gather_chunk(idx_v, out_v):
        def body(g, carry):
            base = g * 16 * UNROLL
            for u in range(UNROLL):
                iv = idx_v[pl.ds(base + u * 16, 16)]
                out_v[pl.ds(base + u * 16, 16)] = plsc.load_gather(row_v, [iv])
            return carry
        lax.fori_loop(0, IC // (16 * UNROLL), body, 0)

    def do_dim(j, tab, idx_hbm, idx_n, out_off):
        pass
        nch = idx_n // IC
        ids = [None, None]
        wds = [None] * nch
        ids[0] = pltpu.async_copy(idx_hbm.at[pl.ds(0, IC)], idx_vs[0],
                                  isems[0])
        for c in range(nch):
            b = c % 2
            if c + 1 < nch:
                ids[(c + 1) % 2] = pltpu.async_copy(
                    idx_hbm.at[pl.ds((c + 1) * IC, IC)],
                    idx_vs[(c + 1) % 2], isems[(c + 1) % 2])
            ids[b].wait()
            if c - 2 >= 0:
                wds[c - 2].wait()
            gather_chunk(idx_vs[b], out_vs[b])
            wds[c] = pltpu.async_copy(
                out_vs[b], out_t.at[j, pl.ds(out_off + c * IC, IC)],
                wsems[b])
        for c in range(max(0, nch - 2), nch):
            wds[c].wait()

    for t in range(DIMS_PER_W):
        j = wid * DIMS_PER_W + t
        do_dim(j, in_tab_t, in_idx_hbm, BATCH, 0)
        do_dim(j, out_tab_t, bc_idx_hbm, N_BC, BATCH)


def kernel(input_words, output_words, noise_words, in_embed_weight,
           out_embed_weight):
    bc_idx = jnp.concatenate(
        [output_words.astype(jnp.int32),
         noise_words.reshape(-1).astype(jnp.int32)], axis=0)
    mesh = plsc.VectorSubcoreMesh(core_axis_name="c", subcore_axis_name="s")
    f = pl.kernel(
        _gather_body,
        mesh=mesh,
        out_type=jax.ShapeDtypeStruct((N_EMBED, TOTAL), jnp.float32),
        scratch_types=[
            pltpu.VMEM((N_VOCAB,), jnp.float32),
            [pltpu.VMEM((IC,), jnp.int32)] * 2,
            [pltpu.VMEM((IC,), jnp.float32)] * 2,
            [pltpu.SemaphoreType.DMA] * 2,
            [pltpu.SemaphoreType.DMA] * 2,
        ],
        compiler_params=pltpu.CompilerParams(use_tc_tiling_on_sc=True,
                                             needs_layout_passes=False),
    )
    out_t = f(
        input_words.astype(jnp.int32),
        bc_idx,
        in_embed_weight.T,
        out_embed_weight.T,
    )
    return out_t.T


# speedup vs baseline: 2.3843x; 1.0426x over previous
import jax
import jax.numpy as jnp
from jax import lax
from jax.experimental import pallas as pl
from jax.experimental.pallas import tpu as pltpu
from jax.experimental.pallas import tpu_sc as plsc

N_VOCAB = 100000
N_EMBED = 64
BATCH = 16384
N_SAMPLES = 5

NC = 2
NS = 16
NW = NC * NS
DIMS_PER_W = N_EMBED // NW

TOTAL = BATCH * (2 + N_SAMPLES)
N_BC = BATCH * (1 + N_SAMPLES)

IC = 4096
UNROLL = 8


def _gather_body(in_idx_hbm, bc_idx_hbm, in_tab_t, out_tab_t, out_t,
                 row_v, idx_vs, out_vs, isems, wsems):
    wid = lax.axis_index("s") * NC + lax.axis_index("c")

    def gather_chunk(idx_v, out_v):
        @plsc.parallel_loop(0, IC, 16, unroll=UNROLL)
        def body(i):
            iv = idx_v[pl.ds(i, 16)]
            out_v[pl.ds(i, 16)] = plsc.load_gather(row_v, [iv])

    def do_dim(j, tab, idx_hbm, idx_n, out_off):
        pltpu.sync_copy(tab.at[j], row_v)
        nch = idx_n // IC
        ids = [None, None]
        wds = [None] * nch
        ids[0] = pltpu.async_copy(idx_hbm.at[pl.ds(0, IC)], idx_vs[0],
                                  isems[0])
        for c in range(nch):
            b = c % 2
            if c + 1 < nch:
                ids[(c + 1) % 2] = pltpu.async_copy(
                    idx_hbm.at[pl.ds((c + 1) * IC, IC)],
                    idx_vs[(c + 1) % 2], isems[(c + 1) % 2])
            ids[b].wait()
            if c - 2 >= 0:
                wds[c - 2].wait()
            gather_chunk(idx_vs[b], out_vs[b])
            wds[c] = pltpu.async_copy(
                out_vs[b], out_t.at[j, pl.ds(out_off + c * IC, IC)],
                wsems[b])
        for c in range(max(0, nch - 2), nch):
            wds[c].wait()

    for t in range(DIMS_PER_W):
        j = wid * DIMS_PER_W + t
        do_dim(j, in_tab_t, in_idx_hbm, BATCH, 0)
        do_dim(j, out_tab_t, bc_idx_hbm, N_BC, BATCH)


def kernel(input_words, output_words, noise_words, in_embed_weight,
           out_embed_weight):
    bc_idx = jnp.concatenate(
        [output_words.astype(jnp.int32),
         noise_words.reshape(-1).astype(jnp.int32)], axis=0)
    mesh = plsc.VectorSubcoreMesh(core_axis_name="c", subcore_axis_name="s")
    f = pl.kernel(
        _gather_body,
        mesh=mesh,
        out_type=jax.ShapeDtypeStruct((N_EMBED, TOTAL), jnp.float32),
        scratch_types=[
            pltpu.VMEM((N_VOCAB,), jnp.float32),
            [pltpu.VMEM((IC,), jnp.int32)] * 2,
            [pltpu.VMEM((IC,), jnp.float32)] * 2,
            [pltpu.SemaphoreType.DMA] * 2,
            [pltpu.SemaphoreType.DMA] * 2,
        ],
        compiler_params=pltpu.CompilerParams(use_tc_tiling_on_sc=True,
                                             needs_layout_passes=False),
    )
    out_t = f(
        input_words.astype(jnp.int32),
        bc_idx,
        in_embed_weight.T,
        out_embed_weight.T,
    )
    return out_t.T
